# Initial kernel scaffold; baseline (speedup 1.0000x reference)
import jax
import jax.numpy as jnp
from jax.experimental import pallas as pl


def kernel(x):
    raise NotImplementedError("write your pallas kernel here")


# speedup vs baseline: 61.7145x; 61.7145x over previous
import jax
import jax.numpy as jnp
from jax.experimental import pallas as pl


def _copy_kernel(x_ref, o_ref):
    o_ref[...] = x_ref[...]


def kernel(x):
    B, T, F = x.shape
    rows = B * T
    x2 = x.reshape(rows, F)
    block_rows = 1024
    out = pl.pallas_call(
        _copy_kernel,
        grid=(rows // block_rows,),
        in_specs=[pl.BlockSpec((block_rows, F), lambda i: (i, 0))],
        out_specs=pl.BlockSpec((block_rows, F), lambda i: (i, 0)),
        out_shape=jax.ShapeDtypeStruct((rows, F), x.dtype),
    )(x2)
    return out.reshape(B, T, F)


# speedup vs baseline: 92.5732x; 1.5000x over previous
import jax
import jax.numpy as jnp
from jax.experimental import pallas as pl


def _copy_kernel(x_ref, o_ref):
    o_ref[...] = x_ref[...]


def kernel(x):
    B, T, F = x.shape
    rows = B * T
    x2 = x.reshape(rows, F)
    block_rows = 4096
    out = pl.pallas_call(
        _copy_kernel,
        grid=(rows // block_rows,),
        in_specs=[pl.BlockSpec((block_rows, F), lambda i: (i, 0))],
        out_specs=pl.BlockSpec((block_rows, F), lambda i: (i, 0)),
        out_shape=jax.ShapeDtypeStruct((rows, F), x.dtype),
    )(x2)
    return out.reshape(B, T, F)


# speedup vs baseline: 95.4101x; 1.0306x over previous
import jax
import jax.numpy as jnp
from jax.experimental import pallas as pl


def _copy_kernel(x_ref, o_ref):
    o_ref[...] = x_ref[...]


def kernel(x):
    B, T, F = x.shape
    rows = B * T
    x2 = x.reshape(rows, F)
    block_rows = 8192
    out = pl.pallas_call(
        _copy_kernel,
        grid=(rows // block_rows,),
        in_specs=[pl.BlockSpec((block_rows, F), lambda i: (i, 0))],
        out_specs=pl.BlockSpec((block_rows, F), lambda i: (i, 0)),
        out_shape=jax.ShapeDtypeStruct((rows, F), x.dtype),
    )(x2)
    return out.reshape(B, T, F)
